# baseline (speedup 1.0000x reference)
import functools

import jax
import jax.numpy as jnp
from jax import lax
from jax.experimental import pallas as pl
from jax.experimental.pallas import tpu as pltpu
from jax.experimental.pallas import tpu_sc as plsc

B = 16384
L = 50
DIM = 64
CHUNK = 128
NSLOT = 5


@functools.cache
def _make_kernel():
    info = plsc.get_sparse_core_info()
    nc, ns = info.num_cores, info.num_subcores
    nw = nc * ns
    b_per_w = B // nw
    cpl = b_per_w // CHUNK
    nchunk = L * cpl
    ngrp = nchunk // NSLOT
    assert ngrp * NSLOT == nchunk
    mesh = plsc.VectorSubcoreMesh(core_axis_name="c", subcore_axis_name="s")

    @functools.partial(
        pl.kernel,
        mesh=mesh,
        out_type=jax.ShapeDtypeStruct((L * 8, B // CHUNK, 8, CHUNK),
                                      jnp.float32),
        scratch_types=[
            pltpu.VMEM((L, b_per_w), jnp.int32),
            pltpu.VMEM((NSLOT, CHUNK, DIM), jnp.float32),
            pltpu.VMEM((NSLOT, 8, 8, CHUNK), jnp.float32),
            pltpu.SemaphoreType.DMA((NSLOT,)),
            pltpu.SemaphoreType.DMA((NSLOT,)),
        ],
        compiler_params=pltpu.CompilerParams(
            use_tc_tiling_on_sc=False, needs_layout_passes=False
        ),
    )
    def k(xt_hbm, table_hbm, out_hbm, idx_v, rows_v, trans_v,
          sem_g, sem_s):
        iota16 = lax.iota(jnp.int32, 16)
        wid = lax.axis_index("s") * nc + lax.axis_index("c")
        b_base = wid * b_per_w
        cb_base = b_base // CHUNK
        pltpu.sync_copy(xt_hbm.at[:, pl.ds(b_base, b_per_w)], idx_v)

        def lbi(g, s):
            c = g * NSLOT + s
            return lax.shift_right_logical(c, 2), jnp.bitwise_and(c, 3)

        def idx_list(g, s):
            l, bi = lbi(g, s)
            return idx_v.at[l, pl.ds(bi * CHUNK, CHUNK)]

        def pidx_gather(g, s):
            pltpu.async_copy(
                table_hbm.at[idx_list(g, s)], rows_v.at[s], sem_g.at[s]
            )

        def gather_wait(g, s):
            pltpu.make_async_copy(
                table_hbm.at[idx_list(g, s)], rows_v.at[s], sem_g.at[s]
            ).wait()

        def out_view(g, s):
            l, bi = lbi(g, s)
            return out_hbm.at[pl.ds(l * 8, 8), cb_base + bi, :, :]

        def store_start(g, s):
            pltpu.async_copy(trans_v.at[s], out_view(g, s), sem_s.at[s])

        def store_wait(g, s):
            pltpu.make_async_copy(
                trans_v.at[s], out_view(g, s), sem_s.at[s]
            ).wait()

        def transpose(g, s):
            l, bi = lbi(g, s)

            @plsc.parallel_loop(0, 8, unroll=2)
            def _(kk):
                for r in range(16):
                    rot = jnp.bitwise_xor(iota16, r)
                    row_v = kk * 16 + rot
                    for dd in range(4):
                        d_v = dd * 16 + iota16
                        vals = plsc.load_gather(
                            rows_v.at[s], [row_v, d_v]
                        )
                        plsc.store_scatter(
                            trans_v.at[s],
                            [lax.shift_right_logical(d_v, 3),
                             jnp.bitwise_and(d_v, 7), row_v],
                            vals,
                        )

        for s in range(NSLOT):
            pidx_gather(0, s)

        @pl.loop(0, ngrp)
        def _(g):
            for s in range(NSLOT):
                gather_wait(g, s)

                @pl.when(g > 0)
                def _w():
                    store_wait(g - 1, s)

                transpose(g, s)
                store_start(g, s)

                @pl.when(g < ngrp - 1)
                def _g():
                    pidx_gather(g + 1, s)

        for s in range(NSLOT):
            store_wait(ngrp - 1, s)

    return k


def kernel(x, table):
    xt = x.T
    out4 = _make_kernel()(xt, table)
    x5 = out4.reshape(L, 8, B // CHUNK, 8, CHUNK)
    return jnp.transpose(x5, (2, 4, 0, 1, 3)).reshape(B, L, DIM)


# speedup vs baseline: 1.2435x; 1.2435x over previous
import functools

import jax
import jax.numpy as jnp
from jax import lax
from jax.experimental import pallas as pl
from jax.experimental.pallas import tpu as pltpu
from jax.experimental.pallas import tpu_sc as plsc

B = 16384
L = 50
DIM = 64
CHUNK = 128
NSLOT = 5


@functools.cache
def _make_kernel():
    info = plsc.get_sparse_core_info()
    nc, ns = info.num_cores, info.num_subcores
    nw = nc * ns
    b_per_w = B // nw
    cpl = b_per_w // CHUNK
    nchunk = L * cpl
    ngrp = nchunk // NSLOT
    assert ngrp * NSLOT == nchunk
    mesh = plsc.VectorSubcoreMesh(core_axis_name="c", subcore_axis_name="s")

    @functools.partial(
        pl.kernel,
        mesh=mesh,
        out_type=jax.ShapeDtypeStruct((L * 8, B // CHUNK, 8, CHUNK),
                                      jnp.float32),
        scratch_types=[
            pltpu.VMEM((L, b_per_w), jnp.int32),
            pltpu.VMEM((NSLOT, CHUNK, DIM), jnp.float32),
            pltpu.VMEM((NSLOT, 8, 8, CHUNK), jnp.float32),
            pltpu.SemaphoreType.DMA((NSLOT,)),
            pltpu.SemaphoreType.DMA((NSLOT,)),
        ],
        compiler_params=pltpu.CompilerParams(
            use_tc_tiling_on_sc=False, needs_layout_passes=False
        ),
    )
    def k(xt_hbm, table_hbm, out_hbm, idx_v, rows_v, trans_v,
          sem_g, sem_s):
        iota16 = lax.iota(jnp.int32, 16)
        wid = lax.axis_index("s") * nc + lax.axis_index("c")
        b_base = wid * b_per_w
        cb_base = b_base // CHUNK
        pltpu.sync_copy(xt_hbm.at[:, pl.ds(b_base, b_per_w)], idx_v)

        def lbi(g, s):
            c = g * NSLOT + s
            return lax.shift_right_logical(c, 2), jnp.bitwise_and(c, 3)

        def idx_list(g, s):
            l, bi = lbi(g, s)
            return idx_v.at[l, pl.ds(bi * CHUNK, CHUNK)]

        def pidx_gather(g, s):
            pltpu.async_copy(
                table_hbm.at[idx_list(g, s)], rows_v.at[s], sem_g.at[s]
            )

        def gather_wait(g, s):
            pltpu.make_async_copy(
                table_hbm.at[idx_list(g, s)], rows_v.at[s], sem_g.at[s]
            ).wait()

        def out_view(g, s):
            l, bi = lbi(g, s)
            return out_hbm.at[pl.ds(l * 8, 8), cb_base + bi, :, :]

        def store_start(g, s):
            pltpu.async_copy(trans_v.at[s], out_view(g, s), sem_s.at[s])

        def store_wait(g, s):
            pltpu.make_async_copy(
                trans_v.at[s], out_view(g, s), sem_s.at[s]
            ).wait()

        def transpose(g, s):
            l, bi = lbi(g, s)

            @plsc.parallel_loop(0, 8)
            def _(kk):
                for r in range(16):
                    rot = jnp.bitwise_xor(iota16, r)
                    row_v = kk * 16 + rot
                    for dd in range(4):
                        d_v = dd * 16 + iota16
                        vals = plsc.load_gather(
                            rows_v.at[s], [row_v, d_v]
                        )
                        plsc.store_scatter(
                            trans_v.at[s],
                            [lax.shift_right_logical(d_v, 3),
                             jnp.bitwise_and(d_v, 7), row_v],
                            vals,
                        )

        for s in range(NSLOT):
            pidx_gather(0, s)

        @pl.loop(0, ngrp)
        def _(g):
            for s in range(NSLOT):
                gather_wait(g, s)

                @pl.when(g > 0)
                def _w():
                    store_wait(g - 1, s)

                transpose(g, s)
                store_start(g, s)

                @pl.when(g < ngrp - 1)
                def _g():
                    pidx_gather(g + 1, s)

        for s in range(NSLOT):
            store_wait(ngrp - 1, s)

    return k


def kernel(x, table):
    xt = x.T
    out4 = _make_kernel()(xt, table)
    x5 = out4.reshape(L, 8, B // CHUNK, 8, CHUNK)
    return jnp.transpose(x5, (2, 4, 0, 1, 3)).reshape(B, L, DIM)


# speedup vs baseline: 1.2536x; 1.0081x over previous
import functools

import jax
import jax.numpy as jnp
from jax import lax
from jax.experimental import pallas as pl
from jax.experimental.pallas import tpu as pltpu
from jax.experimental.pallas import tpu_sc as plsc

B = 16384
L = 50
DIM = 64
CHUNK = 128
NSLOT = 4


@functools.cache
def _make_kernel():
    info = plsc.get_sparse_core_info()
    nc, ns = info.num_cores, info.num_subcores
    nw = nc * ns
    b_per_w = B // nw
    cpl = b_per_w // CHUNK
    assert cpl == NSLOT
    ngrp = L
    mesh = plsc.VectorSubcoreMesh(core_axis_name="c", subcore_axis_name="s")

    @functools.partial(
        pl.kernel,
        mesh=mesh,
        out_type=jax.ShapeDtypeStruct((L * 8, B // CHUNK, 8, CHUNK),
                                      jnp.float32),
        scratch_types=[
            pltpu.VMEM((L, b_per_w), jnp.int32),
            pltpu.VMEM((NSLOT, CHUNK), jnp.int32),
            pltpu.VMEM((NSLOT, CHUNK, DIM), jnp.float32),
            pltpu.VMEM((NSLOT, 8, 8, CHUNK), jnp.float32),
            pltpu.SemaphoreType.DMA((NSLOT,)),
            pltpu.SemaphoreType.DMA((NSLOT,)),
        ],
        compiler_params=pltpu.CompilerParams(
            use_tc_tiling_on_sc=False, needs_layout_passes=False
        ),
    )
    def k(xt_hbm, table_hbm, out_hbm, idx_v, gidx_v, rows_v, trans_v,
          sem_g, sem_s):
        iota16 = lax.iota(jnp.int32, 16)
        wid = lax.axis_index("s") * nc + lax.axis_index("c")
        b_base = wid * b_per_w
        cb_base = b_base // CHUNK
        pltpu.sync_copy(xt_hbm.at[:, pl.ds(b_base, b_per_w)], idx_v)

        def lbi(g, s):
            return g, s

        def pidx_gather(g, s):
            l, bi = lbi(g, s)
            for kk in range(8):
                gidx_v[s, pl.ds(kk * 16, 16)] = (
                    idx_v[l, pl.ds(bi * CHUNK + kk * 16, 16)]
                )
            pltpu.async_copy(
                table_hbm.at[gidx_v.at[s]], rows_v.at[s], sem_g.at[s]
            )

        def gather_wait(s):
            pltpu.make_async_copy(
                table_hbm.at[gidx_v.at[s]], rows_v.at[s], sem_g.at[s]
            ).wait()

        def out_view(g, s):
            l, bi = lbi(g, s)
            return out_hbm.at[pl.ds(l * 8, 8), cb_base + bi, :, :]

        def store_start(g, s):
            pltpu.async_copy(trans_v.at[s], out_view(g, s), sem_s.at[s])

        def store_wait(g, s):
            pltpu.make_async_copy(
                trans_v.at[s], out_view(g, s), sem_s.at[s]
            ).wait()

        def transpose(g, s):
            l, bi = lbi(g, s)

            @plsc.parallel_loop(0, 8)
            def _(kk):
                for r in range(16):
                    rot = jnp.bitwise_xor(iota16, r)
                    row_v = kk * 16 + rot
                    for dd in range(4):
                        d_v = dd * 16 + iota16
                        vals = plsc.load_gather(
                            rows_v.at[s], [row_v, d_v]
                        )
                        plsc.store_scatter(
                            trans_v.at[s],
                            [lax.shift_right_logical(d_v, 3),
                             jnp.bitwise_and(d_v, 7), row_v],
                            vals,
                        )

        for s in range(NSLOT):
            pidx_gather(0, s)

        @pl.loop(0, ngrp)
        def _(g):
            for s in range(NSLOT):
                gather_wait(s)

                @pl.when(g > 0)
                def _w():
                    store_wait(g - 1, s)

                transpose(g, s)
                store_start(g, s)

                @pl.when(g < ngrp - 1)
                def _g():
                    pidx_gather(g + 1, s)

        for s in range(NSLOT):
            store_wait(ngrp - 1, s)

    return k


def kernel(x, table):
    xt = x.T
    out4 = _make_kernel()(xt, table)
    x5 = out4.reshape(L, 8, B // CHUNK, 8, CHUNK)
    return jnp.transpose(x5, (2, 4, 0, 1, 3)).reshape(B, L, DIM)
